# baseline (speedup 1.0000x reference)
import functools

import jax
import jax.numpy as jnp
from jax import lax
from jax.experimental import pallas as pl
from jax.experimental.pallas import tpu as pltpu
from jax.experimental.pallas import tpu_sc as plsc

_NUM_EMB = 1024
_DIM = 64
_ROWS = 18432
_HALF = _ROWS // 2
_TILE = 2048
_GRID = _ROWS // _TILE

_info = plsc.get_sparse_core_info()
_NC, _NS = _info.num_cores, _info.num_subcores
_NW = _NC * _NS
_CH = 128


def _dist_body(z_ref, cb_ref, idx_ref, loss_ref, m_buf, z2_buf):
    i = pl.program_id(0)
    cur = lax.rem(i, 2)
    prev = lax.rem(i + 1, 2)
    cb = cb_ref[...]
    c2 = jnp.sum(cb * cb, axis=1)[None, :]

    mp = m_buf[prev]
    z2p = z2_buf[prev]
    d = (z2p + c2) - 2.0 * mp
    dmin = jnp.min(d, axis=1, keepdims=True)
    j = lax.broadcasted_iota(jnp.int32, (1, _NUM_EMB), 1).astype(jnp.float32)
    idxf = jnp.min(jnp.where(d == dmin, j, jnp.float32(_NUM_EMB)), axis=1)
    idx_ref[...] = idxf.astype(jnp.int32)

    @pl.when(i == 0)
    def _():
        loss_ref[0, 0] = 0.0

    loss_ref[0, 0] += jnp.where(i > 0, jnp.sum(dmin), 0.0)

    zt = z_ref[...]
    m_buf[cur] = lax.dot_general(zt, cb, (((1,), (1,)), ((), ())),
                                 preferred_element_type=jnp.float32)
    z2_buf[cur] = jnp.sum(zt * zt, axis=1, keepdims=True)


def _make_argmin(rows):
    grid = rows // _TILE
    return pl.pallas_call(
        _dist_body,
        grid=(grid + 1,),
        in_specs=[
            pl.BlockSpec((_TILE, _DIM), lambda i: (jnp.minimum(i, grid - 1), 0)),
            pl.BlockSpec((_NUM_EMB, _DIM), lambda i: (0, 0)),
        ],
        out_specs=[
            pl.BlockSpec((_TILE,), lambda i: (jnp.maximum(i, 1) - 1,)),
            pl.BlockSpec((1, 1), lambda i: (0, 0), memory_space=pltpu.SMEM),
        ],
        out_shape=[
            jax.ShapeDtypeStruct((rows,), jnp.int32),
            jax.ShapeDtypeStruct((1, 1), jnp.float32),
        ],
        scratch_shapes=[
            pltpu.VMEM((2, _TILE, _NUM_EMB), jnp.float32),
            pltpu.VMEM((2, _TILE, 1), jnp.float32),
        ],
    )


_sc_mesh = plsc.VectorSubcoreMesh(core_axis_name="c", subcore_axis_name="s")


def _make_gather(rows):
    bpw = rows // _NW
    chunks = []
    off = 0
    while off < bpw:
        sz = min(_CH, bpw - off)
        chunks.append((off, sz))
        off += sz

    @functools.partial(
        pl.kernel,
        mesh=_sc_mesh,
        out_type=jax.ShapeDtypeStruct((rows, 2 * _DIM), jnp.float32),
        scratch_types=[
            pltpu.VMEM((bpw,), jnp.int32),
            pltpu.VMEM((bpw, 2 * _DIM), jnp.float32),
            pltpu.SemaphoreType.DMA,
            pltpu.SemaphoreType.DMA,
        ],
    )
    def _sc_gather(cb_hbm, idx_hbm, out_hbm, idx_v, rows_v, sem, sem_o):
        wid = lax.axis_index("s") * _NC + lax.axis_index("c")
        base = wid * bpw
        pltpu.sync_copy(idx_hbm.at[pl.ds(base, bpw)], idx_v)
        copies = []
        for off, sz in chunks:
            copies.append(
                pltpu.async_copy(
                    cb_hbm.at[idx_v.at[pl.ds(off, sz)]],
                    rows_v.at[pl.ds(off, sz)],
                    sem,
                ))
        outs = []
        for c, (off, sz) in zip(copies, chunks):
            c.wait()
            outs.append(
                pltpu.async_copy(
                    rows_v.at[pl.ds(off, sz)],
                    out_hbm.at[pl.ds(base + off, sz)],
                    sem_o,
                ))
        for o in outs:
            o.wait()

    return _sc_gather


_argmin_full = _make_argmin(_ROWS)
_gather_full = _make_gather(_ROWS)


def kernel(z, codebook):
    zz = z[0]
    z_flat = zz.reshape(-1, zz.shape[-1])
    cb_pad = jnp.pad(codebook, ((0, 0), (0, _DIM)))
    idx, loss_sum = _argmin_full(z_flat, codebook)
    zq = _gather_full(cb_pad, idx)[:, :_DIM]
    m = loss_sum[0, 0] / (_ROWS * _DIM)
    vq_loss = m + 0.1 * m
    return zq.reshape(zz.shape), vq_loss


# speedup vs baseline: 1.0120x; 1.0120x over previous
import functools

import jax
import jax.numpy as jnp
from jax import lax
from jax.experimental import pallas as pl
from jax.experimental.pallas import tpu as pltpu
from jax.experimental.pallas import tpu_sc as plsc

_NUM_EMB = 1024
_DIM = 64
_ROWS = 18432
_HALF = _ROWS // 2
_TILE = 2048
_GRID = _ROWS // _TILE

_info = plsc.get_sparse_core_info()
_NC, _NS = _info.num_cores, _info.num_subcores
_NW = _NC * _NS
_CH = 128


def _dist_body(z_ref, cb_ref, idx_ref, loss_ref, m_buf, z2_buf):
    i = pl.program_id(0)
    cur = lax.rem(i, 2)
    prev = lax.rem(i + 1, 2)
    cb = cb_ref[...]
    c2 = jnp.sum(cb * cb, axis=1)[None, :]

    mp = m_buf[prev]
    z2p = z2_buf[prev]
    d = (z2p + c2) - 2.0 * mp
    dmin = jnp.min(d, axis=1, keepdims=True)
    j = lax.broadcasted_iota(jnp.int32, (1, _NUM_EMB), 1).astype(jnp.float32)
    idxf = jnp.min(jnp.where(d == dmin, j, jnp.float32(_NUM_EMB)), axis=1)
    idx_ref[...] = idxf.astype(jnp.int32)

    @pl.when(i == 0)
    def _():
        loss_ref[0, 0] = 0.0

    loss_ref[0, 0] += jnp.where(i > 0, jnp.sum(dmin), 0.0)

    zt = z_ref[...]
    m_buf[cur] = lax.dot_general(zt, cb, (((1,), (1,)), ((), ())),
                                 preferred_element_type=jnp.float32)
    z2_buf[cur] = jnp.sum(zt * zt, axis=1, keepdims=True)


def _make_argmin(rows):
    grid = rows // _TILE
    return pl.pallas_call(
        _dist_body,
        grid=(grid + 1,),
        in_specs=[
            pl.BlockSpec((_TILE, _DIM), lambda i: (jnp.minimum(i, grid - 1), 0)),
            pl.BlockSpec((_NUM_EMB, _DIM), lambda i: (0, 0)),
        ],
        out_specs=[
            pl.BlockSpec((_TILE,), lambda i: (jnp.maximum(i, 1) - 1,)),
            pl.BlockSpec((1, 1), lambda i: (0, 0), memory_space=pltpu.SMEM),
        ],
        out_shape=[
            jax.ShapeDtypeStruct((rows,), jnp.int32),
            jax.ShapeDtypeStruct((1, 1), jnp.float32),
        ],
        scratch_shapes=[
            pltpu.VMEM((2, _TILE, _NUM_EMB), jnp.float32),
            pltpu.VMEM((2, _TILE, 1), jnp.float32),
        ],
    )


_sc_mesh = plsc.VectorSubcoreMesh(core_axis_name="c", subcore_axis_name="s")


def _make_gather(rows):
    bpw = rows // _NW
    chunks = []
    off = 0
    while off < bpw:
        sz = min(_CH, bpw - off)
        chunks.append((off, sz))
        off += sz

    @functools.partial(
        pl.kernel,
        mesh=_sc_mesh,
        out_type=jax.ShapeDtypeStruct((rows, 2 * _DIM), jnp.float32),
        scratch_types=[
            pltpu.VMEM((bpw,), jnp.int32),
            pltpu.VMEM((bpw, 2 * _DIM), jnp.float32),
            pltpu.SemaphoreType.DMA,
        ],
    )
    def _sc_gather(cb_hbm, idx_hbm, out_hbm, idx_v, rows_v, sem):
        wid = lax.axis_index("s") * _NC + lax.axis_index("c")
        base = wid * bpw
        pltpu.sync_copy(idx_hbm.at[pl.ds(base, bpw)], idx_v)
        copies = []
        for off, sz in chunks:
            copies.append(
                pltpu.async_copy(
                    cb_hbm.at[idx_v.at[pl.ds(off, sz)]],
                    rows_v.at[pl.ds(off, sz)],
                    sem,
                ))
        for c in copies:
            c.wait()
        pltpu.sync_copy(rows_v, out_hbm.at[pl.ds(base, bpw)])

    return _sc_gather


_argmin_full = _make_argmin(_ROWS)
_gather_full = _make_gather(_ROWS)


def kernel(z, codebook):
    zz = z[0]
    z_flat = zz.reshape(-1, zz.shape[-1])
    cb_pad = jnp.pad(codebook, ((0, 0), (0, _DIM)))
    idx, loss_sum = _argmin_full(z_flat, codebook)
    zq = _gather_full(cb_pad, idx)[:, :_DIM]
    m = loss_sum[0, 0] / (_ROWS * _DIM)
    vq_loss = m + 0.1 * m
    return zq.reshape(zz.shape), vq_loss


# speedup vs baseline: 1.0145x; 1.0024x over previous
import functools

import jax
import jax.numpy as jnp
from jax import lax
from jax.experimental import pallas as pl
from jax.experimental.pallas import tpu as pltpu
from jax.experimental.pallas import tpu_sc as plsc

_NUM_EMB = 1024
_DIM = 64
_ROWS = 18432
_TILE = 2048
_GRID = _ROWS // _TILE

_info = plsc.get_sparse_core_info()
_NC, _NS = _info.num_cores, _info.num_subcores
_NW = _NC * _NS
_CH = 128


def _dist_body(z_ref, cb_ref, idx_ref, loss_ref, m_buf, z2_buf):
    i = pl.program_id(0)
    cur = lax.rem(i, 2)
    prev = lax.rem(i + 1, 2)
    cb = cb_ref[...]
    c2 = jnp.sum(cb * cb, axis=1)[None, :]

    mp = m_buf[prev]
    z2p = z2_buf[prev]
    d = (z2p + c2) - 2.0 * mp
    dmin = jnp.min(d, axis=1, keepdims=True)
    j = lax.broadcasted_iota(jnp.int32, (1, _NUM_EMB), 1).astype(jnp.float32)
    idxf = jnp.min(jnp.where(d == dmin, j, jnp.float32(_NUM_EMB)), axis=1)
    idx_ref[...] = idxf.astype(jnp.int32)

    @pl.when(i == 0)
    def _():
        loss_ref[0, 0] = 0.0

    loss_ref[0, 0] += jnp.where(i > 0, jnp.sum(dmin), 0.0)

    zt = z_ref[...]
    m_buf[cur] = lax.dot_general(zt, cb, (((1,), (1,)), ((), ())),
                                 preferred_element_type=jnp.float32)
    z2_buf[cur] = jnp.sum(zt * zt, axis=1, keepdims=True)


def _make_argmin(rows):
    grid = rows // _TILE
    return pl.pallas_call(
        _dist_body,
        grid=(grid + 1,),
        in_specs=[
            pl.BlockSpec((_TILE, _DIM), lambda i: (jnp.minimum(i, grid - 1), 0)),
            pl.BlockSpec((_NUM_EMB, _DIM), lambda i: (0, 0)),
        ],
        out_specs=[
            pl.BlockSpec((_TILE,), lambda i: (jnp.maximum(i, 1) - 1,)),
            pl.BlockSpec((1, 1), lambda i: (0, 0), memory_space=pltpu.SMEM),
        ],
        out_shape=[
            jax.ShapeDtypeStruct((rows,), jnp.int32),
            jax.ShapeDtypeStruct((1, 1), jnp.float32),
        ],
        scratch_shapes=[
            pltpu.VMEM((2, _TILE, _NUM_EMB), jnp.float32),
            pltpu.VMEM((2, _TILE, 1), jnp.float32),
        ],
    )


_sc_mesh = plsc.VectorSubcoreMesh(core_axis_name="c", subcore_axis_name="s")


def _make_gather(rows):
    bpw = rows // _NW
    chunks = []
    off = 0
    while off < bpw:
        sz = min(_CH, bpw - off)
        chunks.append((off, sz))
        off += sz

    @functools.partial(
        pl.kernel,
        mesh=_sc_mesh,
        out_type=jax.ShapeDtypeStruct((rows, 2 * _DIM), jnp.float32),
        scratch_types=[
            pltpu.VMEM((bpw,), jnp.int32),
            pltpu.VMEM((bpw, 2 * _DIM), jnp.float32),
            pltpu.SemaphoreType.DMA,
        ],
    )
    def _sc_gather(cb_hbm, idx_hbm, out_hbm, idx_v, rows_v, sem):
        wid = lax.axis_index("s") * _NC + lax.axis_index("c")
        base = wid * bpw
        pltpu.sync_copy(idx_hbm.at[pl.ds(base, bpw)], idx_v)
        copies = []
        for off, sz in chunks:
            copies.append(
                pltpu.async_copy(
                    cb_hbm.at[idx_v.at[pl.ds(off, sz)]],
                    rows_v.at[pl.ds(off, sz)],
                    sem,
                ))
        for c in copies:
            c.wait()
        pltpu.sync_copy(rows_v, out_hbm.at[pl.ds(base, bpw)])

    return _sc_gather


_argmin_full = _make_argmin(_ROWS)
_gather_full = _make_gather(_ROWS)


def kernel(z, codebook):
    zz = z[0]
    z_flat = zz.reshape(-1, zz.shape[-1])
    cb_pad = jnp.pad(codebook, ((0, 0), (0, _DIM)))
    idx, loss_sum = _argmin_full(z_flat, codebook)
    zq = _gather_full(cb_pad, idx)[:, :_DIM]
    m = loss_sum[0, 0] / (_ROWS * _DIM)
    vq_loss = m + 0.1 * m
    return zq.reshape(zz.shape), vq_loss
